# baseline (speedup 1.0000x reference)
import jax
import jax.numpy as jnp
from jax.experimental import pallas as pl
from jax.experimental.pallas import tpu as pltpu

_NUM_EXPERTS = 8
_TOP_K = 2
_HIDDEN = 768
_T_BLK = 2048


def _router_body(x_ref, wt_ref, b_ref, scores_ref, idx_ref):
    logits = (
        jnp.dot(x_ref[...], wt_ref[...], preferred_element_type=jnp.float32)
        + b_ref[...]
    )
    e_iota = jax.lax.broadcasted_iota(jnp.int32, logits.shape, 1)

    m0 = jnp.max(logits, axis=1, keepdims=True)
    i0 = jnp.min(
        jnp.where(logits == m0, e_iota, _NUM_EXPERTS), axis=1, keepdims=True
    )
    masked = jnp.where(e_iota == i0, -jnp.inf, logits)
    m1 = jnp.max(masked, axis=1, keepdims=True)
    i1 = jnp.min(
        jnp.where(masked == m1, e_iota, _NUM_EXPERTS), axis=1, keepdims=True
    )

    e1 = jnp.exp(m1 - m0)
    p0 = 1.0 / (1.0 + e1)
    p1 = e1 * p0

    scores_ref[...] = jnp.where(
        e_iota == i0, p0, jnp.where(e_iota == i1, p1, 0.0)
    )
    lane = jax.lax.broadcasted_iota(jnp.int32, (logits.shape[0], _TOP_K), 1)
    idx_ref[...] = jnp.where(lane == 0, i0, i1)


def kernel(hidden_states, weight, bias):
    h = hidden_states.reshape(-1, _HIDDEN)
    tokens = h.shape[0]
    wt = weight.T
    b2 = bias.reshape(1, _NUM_EXPERTS)

    grid = (tokens // _T_BLK,)
    scores, indices = pl.pallas_call(
        _router_body,
        grid=grid,
        in_specs=[
            pl.BlockSpec((_T_BLK, _HIDDEN), lambda i: (i, 0)),
            pl.BlockSpec((_HIDDEN, _NUM_EXPERTS), lambda i: (0, 0)),
            pl.BlockSpec((1, _NUM_EXPERTS), lambda i: (0, 0)),
        ],
        out_specs=[
            pl.BlockSpec((_T_BLK, _NUM_EXPERTS), lambda i: (i, 0)),
            pl.BlockSpec((_T_BLK, _TOP_K), lambda i: (i, 0)),
        ],
        out_shape=[
            jax.ShapeDtypeStruct((tokens, _NUM_EXPERTS), jnp.float32),
            jax.ShapeDtypeStruct((tokens, _TOP_K), jnp.int32),
        ],
        compiler_params=pltpu.CompilerParams(
            dimension_semantics=("parallel",)
        ),
    )(h, wt, b2)
    return (scores, indices)


# speedup vs baseline: 1.0733x; 1.0733x over previous
import jax
import jax.numpy as jnp
from jax.experimental import pallas as pl
from jax.experimental.pallas import tpu as pltpu

_NUM_EXPERTS = 8
_TOP_K = 2
_HIDDEN = 768
_T_BLK = 4096


def _router_body(x_ref, wt_ref, b_ref, scores_ref, idx_ref):
    logits = (
        jnp.dot(x_ref[...], wt_ref[...], preferred_element_type=jnp.float32)
        + b_ref[...]
    )
    e_iota = jax.lax.broadcasted_iota(jnp.int32, logits.shape, 1)

    m0 = jnp.max(logits, axis=1, keepdims=True)
    i0 = jnp.min(
        jnp.where(logits == m0, e_iota, _NUM_EXPERTS), axis=1, keepdims=True
    )
    masked = jnp.where(e_iota == i0, -jnp.inf, logits)
    m1 = jnp.max(masked, axis=1, keepdims=True)
    i1 = jnp.min(
        jnp.where(masked == m1, e_iota, _NUM_EXPERTS), axis=1, keepdims=True
    )

    e1 = jnp.exp(m1 - m0)
    p0 = 1.0 / (1.0 + e1)
    p1 = e1 * p0

    scores_ref[...] = jnp.where(
        e_iota == i0, p0, jnp.where(e_iota == i1, p1, 0.0)
    )
    lane = jax.lax.broadcasted_iota(jnp.int32, (logits.shape[0], _TOP_K), 1)
    idx_ref[...] = jnp.where(lane == 0, i0, i1)


def kernel(hidden_states, weight, bias):
    h = hidden_states.reshape(-1, _HIDDEN)
    tokens = h.shape[0]
    wt = weight.T
    b2 = bias.reshape(1, _NUM_EXPERTS)

    grid = (tokens // _T_BLK,)
    scores, indices = pl.pallas_call(
        _router_body,
        grid=grid,
        in_specs=[
            pl.BlockSpec((_T_BLK, _HIDDEN), lambda i: (i, 0)),
            pl.BlockSpec((_HIDDEN, _NUM_EXPERTS), lambda i: (0, 0)),
            pl.BlockSpec((1, _NUM_EXPERTS), lambda i: (0, 0)),
        ],
        out_specs=[
            pl.BlockSpec((_T_BLK, _NUM_EXPERTS), lambda i: (i, 0)),
            pl.BlockSpec((_T_BLK, _TOP_K), lambda i: (i, 0)),
        ],
        out_shape=[
            jax.ShapeDtypeStruct((tokens, _NUM_EXPERTS), jnp.float32),
            jax.ShapeDtypeStruct((tokens, _TOP_K), jnp.int32),
        ],
        compiler_params=pltpu.CompilerParams(
            dimension_semantics=("parallel",)
        ),
    )(h, wt, b2)
    return (scores, indices)
